# baseline (speedup 1.0000x reference)
import functools
import math

import jax
import jax.numpy as jnp
from jax import lax
from jax.experimental import pallas as pl
from jax.experimental.pallas import tpu as pltpu
from jax.experimental.pallas import tpu_sc as plsc

N = 10000
E = 320000
HID = 64
AVG_DEG = 32.0

NC, NS = 2, 16
NW = NC * NS
EPW = E // NW
CH = 80
NCHUNK = EPW // CH
DEP = 5
NR = NCHUNK // DEP
NPC = N // NS

BE = 2000
BN = 1000

f32 = jnp.float32


def _silu(x):
    return x * jax.nn.sigmoid(x)


def _bf(x):
    return x.astype(jnp.bfloat16)


def _dot16(a, b):
    return jnp.dot(_bf(a), _bf(b), preferred_element_type=f32)


@functools.cache
def _sc_kernels():
    mesh = plsc.VectorSubcoreMesh(core_axis_name="c", subcore_axis_name="s")
    cp = pltpu.CompilerParams(use_tc_tiling_on_sc=False)

    gather_scratch = (
        [pltpu.VMEM((EPW,), jnp.int32)] * 2
        + [pltpu.VMEM((CH, HID), f32)] * (2 * DEP)
        + [pltpu.SemaphoreType.DMA] * (4 * DEP)
    )

    @functools.partial(
        pl.kernel,
        compiler_params=cp,
        out_type=(
            jax.ShapeDtypeStruct((E, HID), f32),
            jax.ShapeDtypeStruct((E, HID), f32),
        ),
        mesh=mesh,
        scratch_types=gather_scratch,
    )
    def sc_gather(h, src, dst, gso, gdo, *scr):
        ix_s, ix_d = scr[0], scr[1]
        rs = scr[2:2 + DEP]
        rd = scr[2 + DEP:2 + 2 * DEP]
        sg = scr[2 + 2 * DEP:2 + 3 * DEP]
        dg = scr[2 + 3 * DEP:2 + 4 * DEP]
        sw = scr[2 + 4 * DEP:2 + 5 * DEP]
        dw = scr[2 + 5 * DEP:2 + 6 * DEP]
        wid = lax.axis_index("s") * NC + lax.axis_index("c")
        base = wid * EPW
        pltpu.sync_copy(src.at[pl.ds(base, EPW)], ix_s)
        pltpu.sync_copy(dst.at[pl.ds(base, EPW)], ix_d)

        def fire(i, b):
            pltpu.async_copy(h.at[ix_s.at[pl.ds(i * CH, CH)]], rs[b], sg[b])
            pltpu.async_copy(h.at[ix_d.at[pl.ds(i * CH, CH)]], rd[b], dg[b])

        for b in range(DEP):
            fire(b, b)

        @pl.loop(0, NR)
        def _(j):
            for b in range(DEP):
                i = j * DEP + b
                off = base + i * CH
                pltpu.make_async_copy(
                    h.at[ix_s.at[pl.ds(i * CH, CH)]], rs[b], sg[b]).wait()
                pltpu.async_copy(rs[b], gso.at[pl.ds(off, CH)], sw[b])
                pltpu.make_async_copy(
                    h.at[ix_d.at[pl.ds(i * CH, CH)]], rd[b], dg[b]).wait()
                pltpu.async_copy(rd[b], gdo.at[pl.ds(off, CH)], dw[b])

            @pl.when(j < NR - 1)
            def _():
                for b in range(DEP):
                    i = j * DEP + b
                    off = base + i * CH
                    pltpu.make_async_copy(
                        rs[b], gso.at[pl.ds(off, CH)], sw[b]).wait()
                    pltpu.make_async_copy(
                        rd[b], gdo.at[pl.ds(off, CH)], dw[b]).wait()
                    fire((j + 1) * DEP + b, b)

        for b in range(DEP):
            off = base + ((NR - 1) * DEP + b) * CH
            pltpu.make_async_copy(rs[b], gso.at[pl.ds(off, CH)], sw[b]).wait()
            pltpu.make_async_copy(rd[b], gdo.at[pl.ds(off, CH)], dw[b]).wait()

    scatter_scratch = (
        [pltpu.VMEM((CH,), jnp.int32)] * DEP
        + [pltpu.VMEM((CH, HID), f32)] * DEP
        + [pltpu.SemaphoreType.DMA] * (2 * DEP)
        + [pltpu.VMEM_SHARED((N, HID), f32)]
    )

    @functools.partial(
        pl.kernel,
        compiler_params=cp,
        out_type=jax.ShapeDtypeStruct((NC, N, HID), f32),
        mesh=mesh,
        scratch_types=scatter_scratch,
    )
    def sc_scatter(macc, dstidx, zeros, acc_out, *scr):
        ib = scr[0:DEP]
        pb = scr[DEP:2 * DEP]
        ps = scr[2 * DEP:3 * DEP]
        ss = scr[3 * DEP:4 * DEP]
        acc_sh = scr[4 * DEP]
        cid = lax.axis_index("c")
        sid = lax.axis_index("s")
        wid = sid * NC + cid
        pltpu.sync_copy(zeros.at[pl.ds(sid * NPC, NPC)],
                        acc_sh.at[pl.ds(sid * NPC, NPC)])
        plsc.subcore_barrier()
        base = wid * EPW

        def fire(i, b):
            off = base + i * CH
            pltpu.sync_copy(dstidx.at[pl.ds(off, CH)], ib[b])
            pltpu.async_copy(macc.at[pl.ds(off, CH)], pb[b], ps[b])

        for b in range(DEP):
            fire(b, b)

        @pl.loop(0, NR)
        def _(j):
            for b in range(DEP):
                i = j * DEP + b
                off = base + i * CH
                pltpu.make_async_copy(
                    macc.at[pl.ds(off, CH)], pb[b], ps[b]).wait()
                pltpu.async_copy(pb[b], acc_sh.at[ib[b]], ss[b], add=True)

            @pl.when(j < NR - 1)
            def _():
                for b in range(DEP):
                    pltpu.make_async_copy(
                        pb[b], acc_sh.at[ib[b]], ss[b]).wait()
                    fire((j + 1) * DEP + b, b)

        for b in range(DEP):
            pltpu.make_async_copy(pb[b], acc_sh.at[ib[b]], ss[b]).wait()
        plsc.subcore_barrier()
        pltpu.sync_copy(acc_sh.at[pl.ds(sid * NPC, NPC)],
                        acc_out.at[cid].at[pl.ds(sid * NPC, NPC)])

    return sc_gather, sc_scatter


def _full(shape):
    return pl.BlockSpec(shape, lambda i: tuple(0 for _ in shape))


def _blk(d):
    return pl.BlockSpec((BE, d), lambda i: (i, 0))


def _nblk(d):
    return pl.BlockSpec((BN, d), lambda i: (i, 0))


def _edge_call(first, hs, hd, d2col, d0col, ea16, we1r, be1r, We2, Wm1l,
               bm1r, be2r, Wm2, bm2r, Watt, battr, Wx, bxr):
    grid = (E // BE,)
    win = [_blk(HID), _blk(HID), _blk(1)]
    args = [hs, hd, d2col]
    if first:
        win.append(_blk(1))
        args.append(d0col)
    else:
        win.append(_blk(HID))
        args.append(ea16)
    win += [_full((1, HID)), _full((1, HID)), _full((HID, HID)),
            _full((193, HID)), _full((1, HID)), _full((1, HID)),
            _full((HID, HID)), _full((1, HID)),
            _full((HID, 1)), _full((1, 1)), _full((HID, 1)), _full((1, 1))]
    args += [we1r, be1r, We2, Wm1l, bm1r, be2r, Wm2, bm2r, Watt, battr,
             Wx, bxr]
    out_shape = [jax.ShapeDtypeStruct((E, HID), f32),
                 jax.ShapeDtypeStruct((E, 1), f32)]
    out_specs = [_blk(HID), _blk(1)]
    if first:
        out_shape.append(jax.ShapeDtypeStruct((E, HID), jnp.bfloat16))
        out_specs.append(_blk(HID))

    def body(hsr, hdr, d2r, xtra, we1, be1, We2r, wm1, bm1, be2, wm2, bm2,
             watt, batt, wx, bxr_, mto, coefo, *eaouts):
        d2 = d2r[...]
        if first:
            u = _silu(xtra[...] * we1[...] + be1[...])
            ea = _bf(_dot16(u, We2r[...]) + be2[...])
            eaouts[0][...] = ea
        else:
            ea = xtra[...]
        cat = jnp.concatenate([_bf(hsr[...]), _bf(hdr[...]), _bf(d2), ea],
                              axis=1)
        pre1 = jnp.dot(cat, _bf(wm1[...]),
                       preferred_element_type=f32) + bm1[...]
        m = _silu(_dot16(_silu(pre1), wm2[...]) + bm2[...])
        att = jax.nn.sigmoid(_dot16(m, watt[...]) + batt[...])
        mt = m * att
        coefo[...] = jnp.tanh(_dot16(mt, wx[...]) + bxr_[...])
        mto[...] = mt

    res = pl.pallas_call(
        body, grid=grid, in_specs=win, out_specs=out_specs,
        out_shape=out_shape)(*args)
    return res if first else (res[0], res[1], None)


def _node_call(h, acc0, acc1, Wh1l, bh1r, Wh2, bh2r):
    grid = (N // BN,)
    args = [h, acc0, acc1, Wh1l, bh1r, Wh2, bh2r]
    win = [_nblk(HID), _nblk(HID), _nblk(HID),
           _full((2 * HID, HID)), _full((1, HID)),
           _full((HID, HID)), _full((1, HID))]
    out_shape = jax.ShapeDtypeStruct((N, HID), f32)
    out_specs = _nblk(HID)

    def body(hr, a0, a1, w1, b1, w2, b2, ho):
        agg = a0[...] + a1[...]
        cat = jnp.concatenate([_bf(hr[...]), _bf(agg)], axis=1)
        hh = _silu(jnp.dot(cat, _bf(w1[...]),
                           preferred_element_type=f32) + b1[...])
        ho[...] = hr[...] + _dot16(hh, w2[...]) + b2[...]

    return pl.pallas_call(body, grid=grid, in_specs=win, out_specs=out_specs,
                          out_shape=out_shape)(*args)


def _prep_call(atom_t, nb, isq, tcol, atom_emb, Wn, bnr, Wt1, bt1r,
               Wt2, bt2r, q_emb, Wq, bqr, Win, binr):
    grid = (N // BN,)
    win = [_nblk(1), _nblk(1), _nblk(1), _full((32, 1)),
           _full((100, HID)), _full((HID, HID)), _full((1, HID)),
           _full((32, 32)), _full((1, 32)), _full((32, 32)), _full((1, 32)),
           _full((2, 32)), _full((32, 32)), _full((1, 32)),
           _full((128, HID)), _full((1, HID))]
    out_shape = jax.ShapeDtypeStruct((N, HID), f32)
    out_specs = _nblk(HID)

    def body(ar, nr, qr, tr, emb, wn, bn, wt1, bt1, wt2, bt2, qe, wq, bq,
             wi, bi, h_o):
        P = _dot16(_silu(emb[...]), wn[...])
        oh_a = (ar[...] == lax.broadcasted_iota(jnp.int32, (1, 100), 1))
        h_node = _dot16(oh_a.astype(f32), P) + bn[...]
        freq = jnp.exp(lax.broadcasted_iota(jnp.int32, (1, 16), 1).astype(f32)
                       * (-math.log(10000.0) / 15.0))
        pe = tr[...] * freq
        temb0 = jnp.concatenate([jnp.sin(pe), jnp.cos(pe)], axis=1)
        temb = _dot16(_silu(_dot16(temb0, wt1[...]) + bt1[...]),
                      wt2[...]) + bt2[...]
        oh_b = (nr[...] == lax.broadcasted_iota(jnp.int32, (1, 32), 1))
        temb_node = _dot16(oh_b.astype(f32), temb)
        Q = _dot16(_silu(qe[...]), wq[...])
        isq1 = qr[...] == 1
        qrow = jnp.where(isq1, Q[1:2, :], Q[0:1, :]) + bq[...]
        cat = jnp.concatenate([_bf(h_node), _bf(temb_node), _bf(qrow)],
                              axis=1)
        h_o[...] = jnp.dot(cat, _bf(wi[...]),
                           preferred_element_type=f32) + bi[...]

    return pl.pallas_call(body, grid=grid, in_specs=win, out_specs=out_specs,
                          out_shape=out_shape)(atom_t, nb, isq, tcol,
                                               atom_emb, Wn, bnr, Wt1, bt1r,
                                               Wt2, bt2r, q_emb, Wq, bqr,
                                               Win, binr)


def kernel(pos, atom_emb, Wn, bn, Wt1, bt1, Wt2, bt2, q_emb, Wq, bq, We1,
           be1, We2, be2, Win, bin_, Wm1, bm1, Wm2, bm2, Watt, batt, Wx, bx,
           Wh1, bh1, Wh2, bh2, atom_type, edge_index, node_batch, is_query,
           t):
    L = Wm1.shape[0]
    src = edge_index[0]
    dst = edge_index[1]
    zeros = jnp.zeros((N, HID), f32)
    row = lambda v: v.reshape(1, -1)

    h = _prep_call(
        atom_type.reshape(N, 1), node_batch.reshape(N, 1),
        is_query.reshape(N, 1), t.astype(f32).reshape(32, 1),
        atom_emb, Wn, row(bn), Wt1, row(bt1), Wt2, row(bt2), q_emb, Wq,
        row(bq), Win, row(bin_))

    sc_gather, sc_scatter = _sc_kernels()
    x = pos
    qmask = (is_query == 1).astype(f32)[:, None]
    ea16 = None
    for l in range(L):
        diff = x[src] - x[dst]
        d2 = jnp.sum(diff * diff, axis=-1, keepdims=True)
        d0 = jnp.sqrt(d2 + 1e-8) if l == 0 else None
        hs, hd = sc_gather(h, src, dst)
        mt, coef, ea_new = _edge_call(
            l == 0, hs, hd, d2, d0, ea16, row(We1), row(be1), We2, Wm1[l],
            row(bm1[l]), row(be2), Wm2[l], row(bm2[l]), Watt[l],
            batt[l].reshape(1, 1), Wx[l], bx[l].reshape(1, 1))
        if l == 0:
            ea16 = ea_new
        dx = jax.ops.segment_sum(diff * coef, dst, num_segments=N) / AVG_DEG
        x = x + dx * qmask
        if l < L - 1:
            accs = sc_scatter(mt, dst, zeros)
            h = _node_call(h, accs[0], accs[1], Wh1[l], row(bh1[l]),
                           Wh2[l], row(bh2[l]))
    return x - pos


# speedup vs baseline: 1.1600x; 1.1600x over previous
import functools
import math

import jax
import jax.numpy as jnp
from jax import lax
from jax.experimental import pallas as pl
from jax.experimental.pallas import tpu as pltpu
from jax.experimental.pallas import tpu_sc as plsc

N = 10000
E = 320000
HID = 64
AVG_DEG = 32.0

NC, NS = 2, 16
NW = NC * NS
EPW = E // NW
CH = 80
NCHUNK = EPW // CH
DEP = 5
NR = NCHUNK // DEP
NPC = N // NS

BE = 2000
BN = 1000

f32 = jnp.float32


def _silu(x):
    return x * jax.nn.sigmoid(x)


def _bf(x):
    return x.astype(jnp.bfloat16)


def _dot16(a, b):
    return jnp.dot(_bf(a), _bf(b), preferred_element_type=f32)


@functools.cache
def _sc_kernels():
    mesh = plsc.VectorSubcoreMesh(core_axis_name="c", subcore_axis_name="s")
    cp = pltpu.CompilerParams(use_tc_tiling_on_sc=False)

    gather_scratch = (
        [pltpu.VMEM((EPW,), jnp.int32)] * 2
        + [pltpu.VMEM((CH, HID), f32)] * (2 * DEP)
        + [pltpu.SemaphoreType.DMA] * (4 * DEP)
    )

    @functools.partial(
        pl.kernel,
        compiler_params=cp,
        out_type=(
            jax.ShapeDtypeStruct((E, HID), f32),
            jax.ShapeDtypeStruct((E, HID), f32),
        ),
        mesh=mesh,
        scratch_types=gather_scratch,
    )
    def sc_gather(h, src, dst, gso, gdo, *scr):
        ix_s, ix_d = scr[0], scr[1]
        rs = scr[2:2 + DEP]
        rd = scr[2 + DEP:2 + 2 * DEP]
        sg = scr[2 + 2 * DEP:2 + 3 * DEP]
        dg = scr[2 + 3 * DEP:2 + 4 * DEP]
        sw = scr[2 + 4 * DEP:2 + 5 * DEP]
        dw = scr[2 + 5 * DEP:2 + 6 * DEP]
        wid = lax.axis_index("s") * NC + lax.axis_index("c")
        base = wid * EPW
        pltpu.sync_copy(src.at[pl.ds(base, EPW)], ix_s)
        pltpu.sync_copy(dst.at[pl.ds(base, EPW)], ix_d)

        def fire(i, b):
            pltpu.async_copy(h.at[ix_s.at[pl.ds(i * CH, CH)]], rs[b], sg[b])
            pltpu.async_copy(h.at[ix_d.at[pl.ds(i * CH, CH)]], rd[b], dg[b])

        for b in range(DEP):
            fire(b, b)

        @pl.loop(0, NR)
        def _(j):
            for b in range(DEP):
                i = j * DEP + b
                off = base + i * CH
                pltpu.make_async_copy(
                    h.at[ix_s.at[pl.ds(i * CH, CH)]], rs[b], sg[b]).wait()
                pltpu.async_copy(rs[b], gso.at[pl.ds(off, CH)], sw[b])
                pltpu.make_async_copy(
                    h.at[ix_d.at[pl.ds(i * CH, CH)]], rd[b], dg[b]).wait()
                pltpu.async_copy(rd[b], gdo.at[pl.ds(off, CH)], dw[b])

            @pl.when(j < NR - 1)
            def _():
                for b in range(DEP):
                    i = j * DEP + b
                    off = base + i * CH
                    pltpu.make_async_copy(
                        rs[b], gso.at[pl.ds(off, CH)], sw[b]).wait()
                    pltpu.make_async_copy(
                        rd[b], gdo.at[pl.ds(off, CH)], dw[b]).wait()
                    fire((j + 1) * DEP + b, b)

        for b in range(DEP):
            off = base + ((NR - 1) * DEP + b) * CH
            pltpu.make_async_copy(rs[b], gso.at[pl.ds(off, CH)], sw[b]).wait()
            pltpu.make_async_copy(rd[b], gdo.at[pl.ds(off, CH)], dw[b]).wait()

    def make_scatter(W):
      scatter_scratch = (
        [pltpu.VMEM((CH,), jnp.int32)] * DEP
        + [pltpu.VMEM((CH, W), f32)] * DEP
        + [pltpu.SemaphoreType.DMA] * (2 * DEP)
        + [pltpu.VMEM_SHARED((N, W), f32)]
      )

      @functools.partial(
        pl.kernel,
        compiler_params=cp,
        out_type=jax.ShapeDtypeStruct((NC, N, W), f32),
        mesh=mesh,
        scratch_types=scatter_scratch,
      )
      def sc_scatter(macc, dstidx, zeros, acc_out, *scr):
          ib = scr[0:DEP]
          pb = scr[DEP:2 * DEP]
          ps = scr[2 * DEP:3 * DEP]
          ss = scr[3 * DEP:4 * DEP]
          acc_sh = scr[4 * DEP]
          cid = lax.axis_index("c")
          sid = lax.axis_index("s")
          wid = sid * NC + cid
          pltpu.sync_copy(zeros.at[pl.ds(sid * NPC, NPC)],
                          acc_sh.at[pl.ds(sid * NPC, NPC)])
          plsc.subcore_barrier()
          base = wid * EPW

          def fire(i, b):
              off = base + i * CH
              pltpu.sync_copy(dstidx.at[pl.ds(off, CH)], ib[b])
              pltpu.async_copy(macc.at[pl.ds(off, CH)], pb[b], ps[b])

          for b in range(DEP):
              fire(b, b)

          @pl.loop(0, NR)
          def _(j):
              for b in range(DEP):
                  i = j * DEP + b
                  off = base + i * CH
                  pltpu.make_async_copy(
                      macc.at[pl.ds(off, CH)], pb[b], ps[b]).wait()
                  pltpu.async_copy(pb[b], acc_sh.at[ib[b]], ss[b], add=True)

              @pl.when(j < NR - 1)
              def _():
                  for b in range(DEP):
                      pltpu.make_async_copy(
                          pb[b], acc_sh.at[ib[b]], ss[b]).wait()
                      fire((j + 1) * DEP + b, b)

          for b in range(DEP):
              pltpu.make_async_copy(pb[b], acc_sh.at[ib[b]], ss[b]).wait()
          plsc.subcore_barrier()
          pltpu.sync_copy(acc_sh.at[pl.ds(sid * NPC, NPC)],
                          acc_out.at[cid].at[pl.ds(sid * NPC, NPC)])


      return sc_scatter

    return sc_gather, make_scatter(HID), make_scatter(16)


def _full(shape):
    return pl.BlockSpec(shape, lambda i: tuple(0 for _ in shape))


def _blk(d):
    return pl.BlockSpec((BE, d), lambda i: (i, 0))


def _nblk(d):
    return pl.BlockSpec((BN, d), lambda i: (i, 0))


def _edge_call(first, hs, hd, d2col, d0col, ea16, we1r, be1r, We2, Wm1l,
               bm1r, be2r, Wm2, bm2r, Watt, battr, Wx, bxr):
    grid = (E // BE,)
    win = [_blk(HID), _blk(HID), _blk(1)]
    args = [hs, hd, d2col]
    if first:
        win.append(_blk(1))
        args.append(d0col)
    else:
        win.append(_blk(HID))
        args.append(ea16)
    win += [_full((1, HID)), _full((1, HID)), _full((HID, HID)),
            _full((193, HID)), _full((1, HID)), _full((1, HID)),
            _full((HID, HID)), _full((1, HID)),
            _full((HID, 1)), _full((1, 1)), _full((HID, 1)), _full((1, 1))]
    args += [we1r, be1r, We2, Wm1l, bm1r, be2r, Wm2, bm2r, Watt, battr,
             Wx, bxr]
    out_shape = [jax.ShapeDtypeStruct((E, HID), f32),
                 jax.ShapeDtypeStruct((E, 1), f32)]
    out_specs = [_blk(HID), _blk(1)]
    if first:
        out_shape.append(jax.ShapeDtypeStruct((E, HID), jnp.bfloat16))
        out_specs.append(_blk(HID))

    def body(hsr, hdr, d2r, xtra, we1, be1, We2r, wm1, bm1, be2, wm2, bm2,
             watt, batt, wx, bxr_, mto, coefo, *eaouts):
        d2 = d2r[...]
        if first:
            u = _silu(xtra[...] * we1[...] + be1[...])
            ea = _bf(_dot16(u, We2r[...]) + be2[...])
            eaouts[0][...] = ea
        else:
            ea = xtra[...]
        cat = jnp.concatenate([_bf(hsr[...]), _bf(hdr[...]), _bf(d2), ea],
                              axis=1)
        pre1 = jnp.dot(cat, _bf(wm1[...]),
                       preferred_element_type=f32) + bm1[...]
        m = _silu(_dot16(_silu(pre1), wm2[...]) + bm2[...])
        att = jax.nn.sigmoid(_dot16(m, watt[...]) + batt[...])
        mt = m * att
        coefo[...] = jnp.tanh(_dot16(mt, wx[...]) + bxr_[...])
        mto[...] = mt

    res = pl.pallas_call(
        body, grid=grid, in_specs=win, out_specs=out_specs,
        out_shape=out_shape)(*args)
    return res if first else (res[0], res[1], None)


def _node_call(h, acc0, acc1, Wh1l, bh1r, Wh2, bh2r):
    grid = (N // BN,)
    args = [h, acc0, acc1, Wh1l, bh1r, Wh2, bh2r]
    win = [_nblk(HID), _nblk(HID), _nblk(HID),
           _full((2 * HID, HID)), _full((1, HID)),
           _full((HID, HID)), _full((1, HID))]
    out_shape = jax.ShapeDtypeStruct((N, HID), f32)
    out_specs = _nblk(HID)

    def body(hr, a0, a1, w1, b1, w2, b2, ho):
        agg = a0[...] + a1[...]
        cat = jnp.concatenate([_bf(hr[...]), _bf(agg)], axis=1)
        hh = _silu(jnp.dot(cat, _bf(w1[...]),
                           preferred_element_type=f32) + b1[...])
        ho[...] = hr[...] + _dot16(hh, w2[...]) + b2[...]

    return pl.pallas_call(body, grid=grid, in_specs=win, out_specs=out_specs,
                          out_shape=out_shape)(*args)


def _prep_call(atom_t, nb, isq, tcol, atom_emb, Wn, bnr, Wt1, bt1r,
               Wt2, bt2r, q_emb, Wq, bqr, Win, binr):
    grid = (N // BN,)
    win = [_nblk(1), _nblk(1), _nblk(1), _full((32, 1)),
           _full((100, HID)), _full((HID, HID)), _full((1, HID)),
           _full((32, 32)), _full((1, 32)), _full((32, 32)), _full((1, 32)),
           _full((2, 32)), _full((32, 32)), _full((1, 32)),
           _full((128, HID)), _full((1, HID))]
    out_shape = jax.ShapeDtypeStruct((N, HID), f32)
    out_specs = _nblk(HID)

    def body(ar, nr, qr, tr, emb, wn, bn, wt1, bt1, wt2, bt2, qe, wq, bq,
             wi, bi, h_o):
        P = _dot16(_silu(emb[...]), wn[...])
        oh_a = (ar[...] == lax.broadcasted_iota(jnp.int32, (1, 100), 1))
        h_node = _dot16(oh_a.astype(f32), P) + bn[...]
        freq = jnp.exp(lax.broadcasted_iota(jnp.int32, (1, 16), 1).astype(f32)
                       * (-math.log(10000.0) / 15.0))
        pe = tr[...] * freq
        temb0 = jnp.concatenate([jnp.sin(pe), jnp.cos(pe)], axis=1)
        temb = _dot16(_silu(_dot16(temb0, wt1[...]) + bt1[...]),
                      wt2[...]) + bt2[...]
        oh_b = (nr[...] == lax.broadcasted_iota(jnp.int32, (1, 32), 1))
        temb_node = _dot16(oh_b.astype(f32), temb)
        Q = _dot16(_silu(qe[...]), wq[...])
        isq1 = qr[...] == 1
        qrow = jnp.where(isq1, Q[1:2, :], Q[0:1, :]) + bq[...]
        cat = jnp.concatenate([_bf(h_node), _bf(temb_node), _bf(qrow)],
                              axis=1)
        h_o[...] = jnp.dot(cat, _bf(wi[...]),
                           preferred_element_type=f32) + bi[...]

    return pl.pallas_call(body, grid=grid, in_specs=win, out_specs=out_specs,
                          out_shape=out_shape)(atom_t, nb, isq, tcol,
                                               atom_emb, Wn, bnr, Wt1, bt1r,
                                               Wt2, bt2r, q_emb, Wq, bqr,
                                               Win, binr)


def kernel(pos, atom_emb, Wn, bn, Wt1, bt1, Wt2, bt2, q_emb, Wq, bq, We1,
           be1, We2, be2, Win, bin_, Wm1, bm1, Wm2, bm2, Watt, batt, Wx, bx,
           Wh1, bh1, Wh2, bh2, atom_type, edge_index, node_batch, is_query,
           t):
    L = Wm1.shape[0]
    src = edge_index[0]
    dst = edge_index[1]
    zeros = jnp.zeros((N, HID), f32)
    zeros16 = jnp.zeros((N, 16), f32)
    row = lambda v: v.reshape(1, -1)

    h = _prep_call(
        atom_type.reshape(N, 1), node_batch.reshape(N, 1),
        is_query.reshape(N, 1), t.astype(f32).reshape(32, 1),
        atom_emb, Wn, row(bn), Wt1, row(bt1), Wt2, row(bt2), q_emb, Wq,
        row(bq), Win, row(bin_))

    sc_gather, sc_scatter, sc_scatter16 = _sc_kernels()
    x = pos
    qmask = (is_query == 1).astype(f32)[:, None]
    ea16 = None
    for l in range(L):
        diff = x[src] - x[dst]
        d2 = jnp.sum(diff * diff, axis=-1, keepdims=True)
        d0 = jnp.sqrt(d2 + 1e-8) if l == 0 else None
        hs, hd = sc_gather(h, src, dst)
        mt, coef, ea_new = _edge_call(
            l == 0, hs, hd, d2, d0, ea16, row(We1), row(be1), We2, Wm1[l],
            row(bm1[l]), row(be2), Wm2[l], row(bm2[l]), Watt[l],
            batt[l].reshape(1, 1), Wx[l], bx[l].reshape(1, 1))
        if l == 0:
            ea16 = ea_new
        wd16 = jnp.pad(diff * coef, ((0, 0), (0, 13)))
        a16 = sc_scatter16(wd16, dst, zeros16)
        dx = (a16[0] + a16[1])[:, :3] / AVG_DEG
        x = x + dx * qmask
        if l < L - 1:
            accs = sc_scatter(mt, dst, zeros)
            h = _node_call(h, accs[0], accs[1], Wh1[l], row(bh1[l]),
                           Wh2[l], row(bh2[l]))
    return x - pos
